# baseline (speedup 1.0000x reference)
import jax
import jax.numpy as jnp
from jax import lax
from jax.experimental import pallas as pl
from jax.experimental.pallas import tpu as pltpu
from jax.experimental.pallas import tpu_sc as plsc

D = 64
HID = 128
B = 16384
HIST = 50

NC = 2
NS = 16
NW = NC * NS
BAGS_PER_W = B // NW
CHUNK = 8
NCHUNK = BAGS_PER_W // CHUNK
IDX_PER_CHUNK = CHUNK * HIST
NVEC = IDX_PER_CHUNK // 16


def _pool_body(text_hbm, table_hbm, out_hbm, idx_v, pair_v, rows_v, stage_v,
               sem):
    c = lax.axis_index("c")
    s = lax.axis_index("s")
    wid = s * NC + c

    def chunk_body(ci, carry):
        pltpu.sync_copy(text_hbm.at[wid, ci], idx_v)
        for k in range(NVEC):
            pair_v[pl.ds(k * 16, 16)] = lax.shift_right_logical(
                idx_v[pl.ds(k * 16, 16)], 1)
        pltpu.async_copy(table_hbm.at[pair_v], rows_v, sem).wait()
        for b in range(CHUNK):
            base = b * HIST

            def red(j, accs):
                r = base + j
                pvec = plsc.load_gather(idx_v, [jnp.broadcast_to(r, (16,))])
                m = (pvec & 1) == 1
                return tuple(
                    accs[g] + jnp.where(m,
                                        rows_v[r, pl.ds(64 + g * 16, 16)],
                                        rows_v[r, pl.ds(g * 16, 16)])
                    for g in range(4))

            accs = lax.fori_loop(
                0, HIST, red,
                tuple(jnp.zeros((16,), jnp.float32) for _ in range(4)))
            for g in range(4):
                stage_v[b, pl.ds(g * 16, 16)] = accs[g] * (1.0 / HIST)
        pltpu.sync_copy(
            stage_v,
            out_hbm.at[pl.ds(wid * BAGS_PER_W + ci * CHUNK, CHUNK)])
        return carry

    lax.fori_loop(0, NCHUNK, chunk_body, 0)


def _pool(text3, table2):
    mesh = plsc.VectorSubcoreMesh(core_axis_name="c", subcore_axis_name="s")
    f = pl.kernel(
        _pool_body,
        out_type=jax.ShapeDtypeStruct((B, D), jnp.float32),
        mesh=mesh,
        scratch_types=[
            pltpu.VMEM((IDX_PER_CHUNK,), jnp.int32),
            pltpu.VMEM((IDX_PER_CHUNK,), jnp.int32),
            pltpu.VMEM((IDX_PER_CHUNK, 2 * D), jnp.float32),
            pltpu.VMEM((CHUNK, D), jnp.float32),
            pltpu.SemaphoreType.DMA,
        ],
        compiler_params=pltpu.CompilerParams(needs_layout_passes=False),
    )
    return f(text3, table2)


def _mlp_body(x_ref, w1_ref, b1_ref, w2_ref, b2_ref, o_ref):
    x = x_ref[...]
    h = jnp.dot(x, w1_ref[...], preferred_element_type=jnp.float32) + b1_ref[...]
    h = jnp.where(h > 0, h, jnp.exp(h) - 1.0)
    o = jnp.dot(h, w2_ref[...], preferred_element_type=jnp.float32) + b2_ref[...]
    o_ref[...] = jnp.where(o > 0, o, jnp.exp(o) - 1.0)


def _mlp(x, W1, b1, W2, b2):
    blk = 2048
    return pl.pallas_call(
        _mlp_body,
        grid=(B // blk,),
        in_specs=[
            pl.BlockSpec((blk, D), lambda i: (i, 0)),
            pl.BlockSpec((D, HID), lambda i: (0, 0)),
            pl.BlockSpec((1, HID), lambda i: (0, 0)),
            pl.BlockSpec((HID, D), lambda i: (0, 0)),
            pl.BlockSpec((1, D), lambda i: (0, 0)),
        ],
        out_specs=pl.BlockSpec((blk, D), lambda i: (i, 0)),
        out_shape=jax.ShapeDtypeStruct((B, D), jnp.float32),
    )(x, W1, b1.reshape(1, HID), W2, b2.reshape(1, D))


def kernel(text, emb_table, W1, b1, W2, b2):
    text3 = text.reshape(NW, NCHUNK, IDX_PER_CHUNK).astype(jnp.int32)
    table2 = emb_table.reshape(emb_table.shape[0] // 2, 2 * D)
    pooled = _pool(text3, table2)
    return _mlp(pooled, W1, b1, W2, b2)


# speedup vs baseline: 1.2508x; 1.2508x over previous
import jax
import jax.numpy as jnp
from jax import lax
from jax.experimental import pallas as pl
from jax.experimental.pallas import tpu as pltpu
from jax.experimental.pallas import tpu_sc as plsc

VOCAB = 1000000
D = 64
HID = 128
B = 16384
HIST = 50

NC = 2
NS = 16
NW = NC * NS
BAGS_PER_W = B // NW
CHUNK = 8
NCHUNK = BAGS_PER_W // CHUNK
IDX_PER_CHUNK = CHUNK * HIST


def _pool_body(text_hbm, table_hbm, out_hbm, idx_v, rows_v, stage_v, sem):
    c = lax.axis_index("c")
    s = lax.axis_index("s")
    wid = s * NC + c

    def chunk_body(ci, carry):
        pltpu.sync_copy(text_hbm.at[wid, ci], idx_v)
        pltpu.async_copy(table_hbm.at[idx_v], rows_v, sem).wait()
        for b in range(CHUNK):
            base = b * HIST

            def red(j, accs):
                return tuple(
                    accs[g] + rows_v[base + j, pl.ds(g * 16, 16)]
                    for g in range(4))

            accs = lax.fori_loop(
                0, HIST, red,
                tuple(jnp.zeros((16,), jnp.float32) for _ in range(4)))
            for g in range(4):
                stage_v[b, pl.ds(g * 16, 16)] = accs[g] * (1.0 / HIST)
        pltpu.sync_copy(
            stage_v,
            out_hbm.at[pl.ds(wid * BAGS_PER_W + ci * CHUNK, CHUNK)])
        return carry

    lax.fori_loop(0, NCHUNK, chunk_body, 0)


def _pool(text3, table2):
    mesh = plsc.VectorSubcoreMesh(core_axis_name="c", subcore_axis_name="s")
    f = pl.kernel(
        _pool_body,
        out_type=jax.ShapeDtypeStruct((B, D), jnp.float32),
        mesh=mesh,
        scratch_types=[
            pltpu.VMEM((IDX_PER_CHUNK,), jnp.int32),
            pltpu.VMEM((IDX_PER_CHUNK, 2 * D), jnp.float32),
            pltpu.VMEM((CHUNK, D), jnp.float32),
            pltpu.SemaphoreType.DMA,
        ],
        compiler_params=pltpu.CompilerParams(needs_layout_passes=False),
    )
    return f(text3, table2)


TR_C = 8192
TR_GRID = -(-VOCAB // TR_C)


def _tr_body(i_ref, o_ref):
    x = i_ref[...]
    eye = (lax.broadcasted_iota(jnp.int32, (64, 64), 0)
           == lax.broadcasted_iota(jnp.int32, (64, 64), 1)).astype(jnp.float32)
    y = lax.dot_general(x, eye, (((0,), (0,)), ((), ())),
                        precision=lax.Precision.HIGHEST,
                        preferred_element_type=jnp.float32)
    o_ref[:, 0:64] = y


def _transpose_table(tableT):
    return pl.pallas_call(
        _tr_body,
        grid=(TR_GRID,),
        in_specs=[pl.BlockSpec((64, TR_C), lambda i: (0, i))],
        out_specs=pl.BlockSpec((TR_C, 128), lambda i: (i, 0)),
        out_shape=jax.ShapeDtypeStruct((VOCAB, 128), jnp.float32),
    )(tableT)


def _mlp_body(x_ref, w1_ref, b1_ref, w2_ref, b2_ref, o_ref):
    x = x_ref[...]
    h = jnp.dot(x, w1_ref[...], preferred_element_type=jnp.float32) + b1_ref[...]
    h = jnp.where(h > 0, h, jnp.exp(h) - 1.0)
    o = jnp.dot(h, w2_ref[...], preferred_element_type=jnp.float32) + b2_ref[...]
    o_ref[...] = jnp.where(o > 0, o, jnp.exp(o) - 1.0)


def _mlp(x, W1, b1, W2, b2):
    blk = 2048
    return pl.pallas_call(
        _mlp_body,
        grid=(B // blk,),
        in_specs=[
            pl.BlockSpec((blk, D), lambda i: (i, 0)),
            pl.BlockSpec((D, HID), lambda i: (0, 0)),
            pl.BlockSpec((1, HID), lambda i: (0, 0)),
            pl.BlockSpec((HID, D), lambda i: (0, 0)),
            pl.BlockSpec((1, D), lambda i: (0, 0)),
        ],
        out_specs=pl.BlockSpec((blk, D), lambda i: (i, 0)),
        out_shape=jax.ShapeDtypeStruct((B, D), jnp.float32),
    )(x, W1, b1.reshape(1, HID), W2, b2.reshape(1, D))


def kernel(text, emb_table, W1, b1, W2, b2):
    text3 = text.reshape(NW, NCHUNK, IDX_PER_CHUNK).astype(jnp.int32)
    table2 = _transpose_table(emb_table.T)
    pooled = _pool(text3, table2)
    return _mlp(pooled, W1, b1, W2, b2)


# speedup vs baseline: 1.4938x; 1.1943x over previous
import jax
import jax.numpy as jnp
from jax import lax
from jax.experimental import pallas as pl
from jax.experimental.pallas import tpu as pltpu
from jax.experimental.pallas import tpu_sc as plsc

VOCAB = 1000000
D = 64
HID = 128
B = 16384
HIST = 50

NC = 2
NS = 16
NW = NC * NS
BAGS_PER_W = B // NW
CHUNK = 8
NCHUNK = BAGS_PER_W // CHUNK
IDX_PER_CHUNK = CHUNK * HIST

def _pool_body(text_hbm, table_hbm, out_hbm, idxall_v, rows0_v, rows1_v,
               stage0_v, stage1_v, sem0, sem1):
    c = lax.axis_index("c")
    s = lax.axis_index("s")
    wid = s * NC + c

    pltpu.sync_copy(text_hbm.at[wid], idxall_v)
    pltpu.async_copy(
        table_hbm.at[idxall_v.at[pl.ds(0, IDX_PER_CHUNK)]], rows0_v, sem0)

    bufs = ((rows0_v, stage0_v, sem0), (rows1_v, stage1_v, sem1))

    def pair_body(k2, carry):
        for p in range(2):
            rows_v, stage_v, sem = bufs[p]
            nrows_v, _, nsem = bufs[1 - p]
            ci = 2 * k2 + p
            nx = ci + 1

            @pl.when(nx < NCHUNK)
            def _():
                pltpu.async_copy(
                    table_hbm.at[idxall_v.at[pl.ds(nx * IDX_PER_CHUNK,
                                                  IDX_PER_CHUNK)]],
                    nrows_v, nsem)

            pltpu.make_async_copy(
                table_hbm.at[idxall_v.at[pl.ds(ci * IDX_PER_CHUNK,
                                               IDX_PER_CHUNK)]],
                rows_v, sem).wait()
            for b in range(CHUNK):
                base = b * HIST

                def red(j, accs):
                    return tuple(
                        accs[g] + rows_v[base + j, pl.ds(g * 16, 16)]
                        for g in range(4))

                accs = lax.fori_loop(
                    0, HIST, red,
                    tuple(jnp.zeros((16,), jnp.float32) for _ in range(4)))
                for g in range(4):
                    stage_v[b, pl.ds(g * 16, 16)] = accs[g] * (1.0 / HIST)
            pltpu.sync_copy(
                stage_v,
                out_hbm.at[pl.ds(wid * BAGS_PER_W + ci * CHUNK, CHUNK)])
        return carry

    lax.fori_loop(0, NCHUNK // 2, pair_body, 0)


def _pool(text3, table2):
    mesh = plsc.VectorSubcoreMesh(core_axis_name="c", subcore_axis_name="s")
    f = pl.kernel(
        _pool_body,
        out_type=jax.ShapeDtypeStruct((B, D), jnp.float32),
        mesh=mesh,
        scratch_types=[
            pltpu.VMEM((NCHUNK * IDX_PER_CHUNK,), jnp.int32),
            pltpu.VMEM((IDX_PER_CHUNK, 128), jnp.float32),
            pltpu.VMEM((IDX_PER_CHUNK, 128), jnp.float32),
            pltpu.VMEM((CHUNK, D), jnp.float32),
            pltpu.VMEM((CHUNK, D), jnp.float32),
            pltpu.SemaphoreType.DMA,
            pltpu.SemaphoreType.DMA,
        ],
        compiler_params=pltpu.CompilerParams(needs_layout_passes=False),
    )
    return f(text3, table2)


TR_C = 8192
TR_GRID = -(-VOCAB // TR_C)


def _tr_body(i_ref, o_ref):
    x = i_ref[...]
    eye = (lax.broadcasted_iota(jnp.int32, (64, 64), 0)
           == lax.broadcasted_iota(jnp.int32, (64, 64), 1)).astype(jnp.float32)
    y = lax.dot_general(x, eye, (((0,), (0,)), ((), ())),
                        precision=lax.Precision.HIGHEST,
                        preferred_element_type=jnp.float32)
    o_ref[:, 0:64] = y


def _transpose_table(tableT):
    return pl.pallas_call(
        _tr_body,
        grid=(TR_GRID,),
        in_specs=[pl.BlockSpec((64, TR_C), lambda i: (0, i))],
        out_specs=pl.BlockSpec((TR_C, 128), lambda i: (i, 0)),
        out_shape=jax.ShapeDtypeStruct((VOCAB, 128), jnp.float32),
    )(tableT)


def _mlp_body(x_ref, w1_ref, b1_ref, w2_ref, b2_ref, o_ref):
    x = x_ref[...]
    h = jnp.dot(x, w1_ref[...], preferred_element_type=jnp.float32) + b1_ref[...]
    h = jnp.where(h > 0, h, jnp.exp(h) - 1.0)
    o = jnp.dot(h, w2_ref[...], preferred_element_type=jnp.float32) + b2_ref[...]
    o_ref[...] = jnp.where(o > 0, o, jnp.exp(o) - 1.0)


def _mlp(x, W1, b1, W2, b2):
    blk = 2048
    return pl.pallas_call(
        _mlp_body,
        grid=(B // blk,),
        in_specs=[
            pl.BlockSpec((blk, D), lambda i: (i, 0)),
            pl.BlockSpec((D, HID), lambda i: (0, 0)),
            pl.BlockSpec((1, HID), lambda i: (0, 0)),
            pl.BlockSpec((HID, D), lambda i: (0, 0)),
            pl.BlockSpec((1, D), lambda i: (0, 0)),
        ],
        out_specs=pl.BlockSpec((blk, D), lambda i: (i, 0)),
        out_shape=jax.ShapeDtypeStruct((B, D), jnp.float32),
    )(x, W1, b1.reshape(1, HID), W2, b2.reshape(1, D))


def kernel(text, emb_table, W1, b1, W2, b2):
    text3 = text.reshape(NW, NCHUNK * IDX_PER_CHUNK).astype(jnp.int32)
    table2 = _transpose_table(emb_table.T)
    pooled = _pool(text3, table2)
    return _mlp(pooled, W1, b1, W2, b2)


# speedup vs baseline: 2.1001x; 1.4059x over previous
import jax
import jax.numpy as jnp
from jax import lax
from jax.experimental import pallas as pl
from jax.experimental.pallas import tpu as pltpu
from jax.experimental.pallas import tpu_sc as plsc

VOCAB = 1000000
D = 64
HID = 128
B = 16384
HIST = 50

NC = 2
NS = 16
NW = NC * NS
BAGS_PER_W = B // NW
CHUNK = 8
NCHUNK = BAGS_PER_W // CHUNK
IDX_PER_CHUNK = CHUNK * HIST

def _pool_body(text_hbm, table_hbm, out_hbm, idxall_v, rows0_v, rows1_v,
               stage0_v, stage1_v, sem0, sem1):
    c = lax.axis_index("c")
    s = lax.axis_index("s")
    wid = s * NC + c

    pltpu.sync_copy(text_hbm.at[wid], idxall_v)
    pltpu.async_copy(
        table_hbm.at[idxall_v.at[pl.ds(0, IDX_PER_CHUNK)]], rows0_v, sem0)

    bufs = ((rows0_v, stage0_v, sem0), (rows1_v, stage1_v, sem1))

    def pair_body(k2, carry):
        for p in range(2):
            rows_v, stage_v, sem = bufs[p]
            nrows_v, _, nsem = bufs[1 - p]
            ci = 2 * k2 + p
            nx = ci + 1

            @pl.when(nx < NCHUNK)
            def _():
                pltpu.async_copy(
                    table_hbm.at[idxall_v.at[pl.ds(nx * IDX_PER_CHUNK,
                                                  IDX_PER_CHUNK)]],
                    nrows_v, nsem)

            pltpu.make_async_copy(
                table_hbm.at[idxall_v.at[pl.ds(ci * IDX_PER_CHUNK,
                                               IDX_PER_CHUNK)]],
                rows_v, sem).wait()
            for b in range(CHUNK):
                base = b * HIST

                def red(j, accs):
                    return tuple(
                        accs[g] + rows_v[base + j, pl.ds(g * 16, 16)]
                        for g in range(4))

                accs = lax.fori_loop(
                    0, HIST, red,
                    tuple(jnp.zeros((16,), jnp.float32) for _ in range(4)))
                for g in range(4):
                    stage_v[b, pl.ds(g * 16, 16)] = accs[g] * (1.0 / HIST)
            pltpu.sync_copy(
                stage_v,
                out_hbm.at[pl.ds(wid * BAGS_PER_W + ci * CHUNK, CHUNK)])
        return carry

    lax.fori_loop(0, NCHUNK // 2, pair_body, 0)


def _pool(text3, table2):
    mesh = plsc.VectorSubcoreMesh(core_axis_name="c", subcore_axis_name="s")
    f = pl.kernel(
        _pool_body,
        out_type=jax.ShapeDtypeStruct((B, D), jnp.float32),
        mesh=mesh,
        scratch_types=[
            pltpu.VMEM((NCHUNK * IDX_PER_CHUNK,), jnp.int32),
            pltpu.VMEM((IDX_PER_CHUNK, 128), jnp.float32),
            pltpu.VMEM((IDX_PER_CHUNK, 128), jnp.float32),
            pltpu.VMEM((CHUNK, D), jnp.float32),
            pltpu.VMEM((CHUNK, D), jnp.float32),
            pltpu.SemaphoreType.DMA,
            pltpu.SemaphoreType.DMA,
        ],
        compiler_params=pltpu.CompilerParams(needs_layout_passes=False),
    )
    return f(text3, table2)


TR_C = 16384
TR_GRID = -(-VOCAB // TR_C)


def _tr_body(i_ref, o_ref):
    x = i_ref[...]
    eye = (lax.broadcasted_iota(jnp.int32, (64, 64), 0)
           == lax.broadcasted_iota(jnp.int32, (64, 64), 1)).astype(jnp.float32)
    y = lax.dot_general(x, eye, (((0,), (0,)), ((), ())),
                        preferred_element_type=jnp.float32)
    o_ref[:, 0:64] = y


def _transpose_table(tableT):
    return pl.pallas_call(
        _tr_body,
        grid=(TR_GRID,),
        in_specs=[pl.BlockSpec((64, TR_C), lambda i: (0, i))],
        out_specs=pl.BlockSpec((TR_C, 128), lambda i: (i, 0)),
        out_shape=jax.ShapeDtypeStruct((VOCAB, 128), jnp.float32),
    )(tableT)


def _mlp_body(x_ref, w1_ref, b1_ref, w2_ref, b2_ref, o_ref):
    x = x_ref[...]
    h = jnp.dot(x, w1_ref[...], preferred_element_type=jnp.float32) + b1_ref[...]
    h = jnp.where(h > 0, h, jnp.exp(h) - 1.0)
    o = jnp.dot(h, w2_ref[...], preferred_element_type=jnp.float32) + b2_ref[...]
    o_ref[...] = jnp.where(o > 0, o, jnp.exp(o) - 1.0)


def _mlp(x, W1, b1, W2, b2):
    blk = 2048
    return pl.pallas_call(
        _mlp_body,
        grid=(B // blk,),
        in_specs=[
            pl.BlockSpec((blk, D), lambda i: (i, 0)),
            pl.BlockSpec((D, HID), lambda i: (0, 0)),
            pl.BlockSpec((1, HID), lambda i: (0, 0)),
            pl.BlockSpec((HID, D), lambda i: (0, 0)),
            pl.BlockSpec((1, D), lambda i: (0, 0)),
        ],
        out_specs=pl.BlockSpec((blk, D), lambda i: (i, 0)),
        out_shape=jax.ShapeDtypeStruct((B, D), jnp.float32),
    )(x, W1, b1.reshape(1, HID), W2, b2.reshape(1, D))


def kernel(text, emb_table, W1, b1, W2, b2):
    text3 = text.reshape(NW, NCHUNK * IDX_PER_CHUNK).astype(jnp.int32)
    table2 = _transpose_table(emb_table.T)
    pooled = _pool(text3, table2)
    return _mlp(pooled, W1, b1, W2, b2)


# speedup vs baseline: 2.1087x; 1.0041x over previous
import jax
import jax.numpy as jnp
from jax import lax
from jax.experimental import pallas as pl
from jax.experimental.pallas import tpu as pltpu
from jax.experimental.pallas import tpu_sc as plsc

VOCAB = 1000000
D = 64
HID = 128
B = 16384
HIST = 50

NC = 2
NS = 16
NW = NC * NS
BAGS_PER_W = B // NW
CHUNK = 8
NCHUNK = BAGS_PER_W // CHUNK
IDX_PER_CHUNK = CHUNK * HIST

def _pool_body(text_hbm, table_hbm, out_hbm, idxall_v, rows0_v, rows1_v,
               stage0_v, stage1_v, sem0, sem1):
    c = lax.axis_index("c")
    s = lax.axis_index("s")
    wid = s * NC + c

    pltpu.sync_copy(text_hbm.at[wid], idxall_v)
    pltpu.async_copy(
        table_hbm.at[idxall_v.at[pl.ds(0, IDX_PER_CHUNK)]], rows0_v, sem0)

    bufs = ((rows0_v, stage0_v, sem0), (rows1_v, stage1_v, sem1))

    def pair_body(k2, carry):
        for p in range(2):
            rows_v, stage_v, sem = bufs[p]
            nrows_v, _, nsem = bufs[1 - p]
            ci = 2 * k2 + p
            nx = ci + 1

            @pl.when(nx < NCHUNK)
            def _():
                pltpu.async_copy(
                    table_hbm.at[idxall_v.at[pl.ds(nx * IDX_PER_CHUNK,
                                                  IDX_PER_CHUNK)]],
                    nrows_v, nsem)

            pltpu.make_async_copy(
                table_hbm.at[idxall_v.at[pl.ds(ci * IDX_PER_CHUNK,
                                               IDX_PER_CHUNK)]],
                rows_v, sem).wait()
            for b in range(CHUNK):
                base = b * HIST

                def red(j, accs):
                    return tuple(
                        accs[g] + rows_v[base + j, pl.ds(g * 16, 16)]
                        for g in range(4))

                accs = lax.fori_loop(
                    0, HIST, red,
                    tuple(jnp.zeros((16,), jnp.float32) for _ in range(4)))
                for g in range(4):
                    stage_v[b, pl.ds(g * 16, 16)] = accs[g] * (1.0 / HIST)
            pltpu.sync_copy(
                stage_v,
                out_hbm.at[pl.ds(wid * BAGS_PER_W + ci * CHUNK, CHUNK)])
        return carry

    lax.fori_loop(0, NCHUNK // 2, pair_body, 0)


def _pool(text3, table2):
    mesh = plsc.VectorSubcoreMesh(core_axis_name="c", subcore_axis_name="s")
    f = pl.kernel(
        _pool_body,
        out_type=jax.ShapeDtypeStruct((B, D), jnp.float32),
        mesh=mesh,
        scratch_types=[
            pltpu.VMEM((NCHUNK * IDX_PER_CHUNK,), jnp.int32),
            pltpu.VMEM((IDX_PER_CHUNK, 128), jnp.float32),
            pltpu.VMEM((IDX_PER_CHUNK, 128), jnp.float32),
            pltpu.VMEM((CHUNK, D), jnp.float32),
            pltpu.VMEM((CHUNK, D), jnp.float32),
            pltpu.SemaphoreType.DMA,
            pltpu.SemaphoreType.DMA,
        ],
        compiler_params=pltpu.CompilerParams(needs_layout_passes=False),
    )
    return f(text3, table2)


TR_C = 16384
TR_GRID = -(-VOCAB // TR_C)


def _tr_body(i_ref, o_ref):
    x = i_ref[...]
    eye = (lax.broadcasted_iota(jnp.int32, (64, 64), 0)
           == lax.broadcasted_iota(jnp.int32, (64, 64), 1)).astype(jnp.float32)
    y = lax.dot_general(x, eye, (((0,), (0,)), ((), ())),
                        preferred_element_type=jnp.float32)
    o_ref[:, 0:64] = y


def _transpose_table(tableT):
    return pl.pallas_call(
        _tr_body,
        grid=(TR_GRID,),
        in_specs=[pl.BlockSpec((64, TR_C), lambda i: (0, i))],
        out_specs=pl.BlockSpec((TR_C, 128), lambda i: (i, 0)),
        out_shape=jax.ShapeDtypeStruct((VOCAB, 128), jnp.float32),
    )(tableT)


def _mlp_body(x_ref, w1_ref, b1_ref, w2_ref, b2_ref, o_ref):
    x = x_ref[...]
    h = jnp.dot(x, w1_ref[...], preferred_element_type=jnp.float32) + b1_ref[...]
    h = jnp.where(h > 0, h, jnp.exp(h) - 1.0)
    o = jnp.dot(h, w2_ref[...], preferred_element_type=jnp.float32) + b2_ref[...]
    o = jnp.where(o > 0, o, jnp.exp(o) - 1.0)
    eye = (lax.broadcasted_iota(jnp.int32, (D, D), 0)
           == lax.broadcasted_iota(jnp.int32, (D, D), 1)).astype(jnp.float32)
    o_ref[...] = lax.dot_general(eye, o, (((0,), (1,)), ((), ())),
                                 precision=lax.Precision.HIGHEST,
                                 preferred_element_type=jnp.float32)


def _mlp(x, W1, b1, W2, b2):
    blk = 2048
    outT = pl.pallas_call(
        _mlp_body,
        grid=(B // blk,),
        in_specs=[
            pl.BlockSpec((blk, D), lambda i: (i, 0)),
            pl.BlockSpec((D, HID), lambda i: (0, 0)),
            pl.BlockSpec((1, HID), lambda i: (0, 0)),
            pl.BlockSpec((HID, D), lambda i: (0, 0)),
            pl.BlockSpec((1, D), lambda i: (0, 0)),
        ],
        out_specs=pl.BlockSpec((D, blk), lambda i: (0, i)),
        out_shape=jax.ShapeDtypeStruct((D, B), jnp.float32),
    )(x, W1, b1.reshape(1, HID), W2, b2.reshape(1, D))
    return outT.T


def kernel(text, emb_table, W1, b1, W2, b2):
    text3 = text.reshape(NW, NCHUNK * IDX_PER_CHUNK).astype(jnp.int32)
    table2 = _transpose_table(emb_table.T)
    pooled = _pool(text3, table2)
    return _mlp(pooled, W1, b1, W2, b2)


# speedup vs baseline: 2.1407x; 1.0152x over previous
import jax
import jax.numpy as jnp
from jax import lax
from jax.experimental import pallas as pl
from jax.experimental.pallas import tpu as pltpu
from jax.experimental.pallas import tpu_sc as plsc

VOCAB = 1000000
D = 64
HID = 128
B = 16384
HIST = 50

NC = 2
NS = 16
NW = NC * NS
BAGS_PER_W = B // NW
CHUNK = 8
NCHUNK = BAGS_PER_W // CHUNK
IDX_PER_CHUNK = CHUNK * HIST

def _pool_body(text_hbm, table_hbm, out_hbm, idxall_v, rows0_v, rows1_v,
               stage0_v, stage1_v, sem0, sem1):
    c = lax.axis_index("c")
    s = lax.axis_index("s")
    wid = s * NC + c

    pltpu.sync_copy(text_hbm.at[wid], idxall_v)
    pltpu.async_copy(
        table_hbm.at[idxall_v.at[pl.ds(0, IDX_PER_CHUNK)]], rows0_v, sem0)

    bufs = ((rows0_v, stage0_v, sem0), (rows1_v, stage1_v, sem1))

    def pair_body(k2, carry):
        for p in range(2):
            rows_v, stage_v, sem = bufs[p]
            nrows_v, _, nsem = bufs[1 - p]
            ci = 2 * k2 + p
            nx = ci + 1

            @pl.when(nx < NCHUNK)
            def _():
                pltpu.async_copy(
                    table_hbm.at[idxall_v.at[pl.ds(nx * IDX_PER_CHUNK,
                                                  IDX_PER_CHUNK)]],
                    nrows_v, nsem)

            pltpu.make_async_copy(
                table_hbm.at[idxall_v.at[pl.ds(ci * IDX_PER_CHUNK,
                                               IDX_PER_CHUNK)]],
                rows_v, sem).wait()
            for b in range(CHUNK):
                base = b * HIST

                def red(j, accs):
                    return tuple(
                        accs[g] + rows_v[base + j, pl.ds(g * 16, 16)]
                        for g in range(4))

                accs = lax.fori_loop(
                    0, HIST, red,
                    tuple(jnp.zeros((16,), jnp.float32) for _ in range(4)))
                for g in range(4):
                    stage_v[b, pl.ds(g * 16, 16)] = accs[g] * (1.0 / HIST)
            pltpu.sync_copy(
                stage_v,
                out_hbm.at[pl.ds(wid * BAGS_PER_W + ci * CHUNK, CHUNK)])
        return carry

    lax.fori_loop(0, NCHUNK // 2, pair_body, 0)


def _pool(text3, table2):
    mesh = plsc.VectorSubcoreMesh(core_axis_name="c", subcore_axis_name="s")
    f = pl.kernel(
        _pool_body,
        out_type=jax.ShapeDtypeStruct((B, D), jnp.float32),
        mesh=mesh,
        scratch_types=[
            pltpu.VMEM((NCHUNK * IDX_PER_CHUNK,), jnp.int32),
            pltpu.VMEM((IDX_PER_CHUNK, 128), jnp.float32),
            pltpu.VMEM((IDX_PER_CHUNK, 128), jnp.float32),
            pltpu.VMEM((CHUNK, D), jnp.float32),
            pltpu.VMEM((CHUNK, D), jnp.float32),
            pltpu.SemaphoreType.DMA,
            pltpu.SemaphoreType.DMA,
        ],
        compiler_params=pltpu.CompilerParams(needs_layout_passes=False),
    )
    return f(text3, table2)


TR_C = 32768
TR_GRID = -(-VOCAB // TR_C)


def _tr_body(i_ref, o_ref):
    x = i_ref[...]
    eye = (lax.broadcasted_iota(jnp.int32, (64, 64), 0)
           == lax.broadcasted_iota(jnp.int32, (64, 64), 1)).astype(jnp.float32)
    y = lax.dot_general(x, eye, (((0,), (0,)), ((), ())),
                        preferred_element_type=jnp.float32)
    o_ref[:, 0:64] = y


def _transpose_table(tableT):
    return pl.pallas_call(
        _tr_body,
        grid=(TR_GRID,),
        in_specs=[pl.BlockSpec((64, TR_C), lambda i: (0, i))],
        out_specs=pl.BlockSpec((TR_C, 128), lambda i: (i, 0)),
        out_shape=jax.ShapeDtypeStruct((VOCAB, 128), jnp.float32),
    )(tableT)


def _mlp_body(x_ref, w1_ref, b1_ref, w2_ref, b2_ref, o_ref):
    x = x_ref[...]
    h = jnp.dot(x, w1_ref[...], preferred_element_type=jnp.float32) + b1_ref[...]
    h = jnp.where(h > 0, h, jnp.exp(h) - 1.0)
    o = jnp.dot(h, w2_ref[...], preferred_element_type=jnp.float32) + b2_ref[...]
    o = jnp.where(o > 0, o, jnp.exp(o) - 1.0)
    eye = (lax.broadcasted_iota(jnp.int32, (D, D), 0)
           == lax.broadcasted_iota(jnp.int32, (D, D), 1)).astype(jnp.float32)
    o_ref[...] = lax.dot_general(eye, o, (((0,), (1,)), ((), ())),
                                 precision=lax.Precision.HIGHEST,
                                 preferred_element_type=jnp.float32)


def _mlp(x, W1, b1, W2, b2):
    blk = 2048
    outT = pl.pallas_call(
        _mlp_body,
        grid=(B // blk,),
        in_specs=[
            pl.BlockSpec((blk, D), lambda i: (i, 0)),
            pl.BlockSpec((D, HID), lambda i: (0, 0)),
            pl.BlockSpec((1, HID), lambda i: (0, 0)),
            pl.BlockSpec((HID, D), lambda i: (0, 0)),
            pl.BlockSpec((1, D), lambda i: (0, 0)),
        ],
        out_specs=pl.BlockSpec((D, blk), lambda i: (0, i)),
        out_shape=jax.ShapeDtypeStruct((D, B), jnp.float32),
    )(x, W1, b1.reshape(1, HID), W2, b2.reshape(1, D))
    return outT.T


def kernel(text, emb_table, W1, b1, W2, b2):
    text3 = text.reshape(NW, NCHUNK * IDX_PER_CHUNK).astype(jnp.int32)
    table2 = _transpose_table(emb_table.T)
    pooled = _pool(text3, table2)
    return _mlp(pooled, W1, b1, W2, b2)


# speedup vs baseline: 2.2850x; 1.0674x over previous
import jax
import jax.numpy as jnp
from jax import lax
from jax.experimental import pallas as pl
from jax.experimental.pallas import tpu as pltpu
from jax.experimental.pallas import tpu_sc as plsc

VOCAB = 1000000
D = 64
HID = 128
B = 16384
HIST = 50

NC = 2
NS = 16
NW = NC * NS
BAGS_PER_W = B // NW
CHUNK = 4
NCHUNK = BAGS_PER_W // CHUNK
IDX_PER_CHUNK = CHUNK * HIST
NBUF = 4

def _pool_body(text_hbm, table_hbm, out_hbm, idxall_v, *rest):
    rows_bufs = rest[0:NBUF]
    stage_bufs = rest[NBUF:2 * NBUF]
    sems = rest[2 * NBUF:3 * NBUF]
    c = lax.axis_index("c")
    s = lax.axis_index("s")
    wid = s * NC + c

    def _gather(ci, rows_v, sem):
        return pltpu.async_copy(
            table_hbm.at[idxall_v.at[pl.ds(ci * IDX_PER_CHUNK,
                                           IDX_PER_CHUNK)]],
            rows_v, sem)

    pltpu.sync_copy(text_hbm.at[wid], idxall_v)
    for p in range(NBUF - 1):
        _gather(p, rows_bufs[p], sems[p])

    def ring_body(k, carry):
        for p in range(NBUF):
            rows_v, stage_v, sem = rows_bufs[p], stage_bufs[p], sems[p]
            ci = NBUF * k + p
            nx = ci + NBUF - 1

            @pl.when(nx < NCHUNK)
            def _():
                _gather(nx, rows_bufs[(p + NBUF - 1) % NBUF],
                        sems[(p + NBUF - 1) % NBUF])

            pltpu.make_async_copy(
                table_hbm.at[idxall_v.at[pl.ds(ci * IDX_PER_CHUNK,
                                               IDX_PER_CHUNK)]],
                rows_v, sem).wait()
            for b in range(CHUNK):
                base = b * HIST

                def red(j, accs):
                    return tuple(
                        accs[g] + rows_v[base + j, pl.ds(g * 16, 16)]
                        for g in range(4))

                accs = lax.fori_loop(
                    0, HIST, red,
                    tuple(jnp.zeros((16,), jnp.float32) for _ in range(4)))
                for g in range(4):
                    stage_v[b, pl.ds(g * 16, 16)] = accs[g] * (1.0 / HIST)
            pltpu.sync_copy(
                stage_v,
                out_hbm.at[pl.ds(wid * BAGS_PER_W + ci * CHUNK, CHUNK)])
        return carry

    lax.fori_loop(0, NCHUNK // NBUF, ring_body, 0)


def _pool(text3, table2):
    mesh = plsc.VectorSubcoreMesh(core_axis_name="c", subcore_axis_name="s")
    f = pl.kernel(
        _pool_body,
        out_type=jax.ShapeDtypeStruct((B, D), jnp.float32),
        mesh=mesh,
        scratch_types=(
            [pltpu.VMEM((NCHUNK * IDX_PER_CHUNK,), jnp.int32)]
            + [pltpu.VMEM((IDX_PER_CHUNK, 128), jnp.float32)] * NBUF
            + [pltpu.VMEM((CHUNK, D), jnp.float32)] * NBUF
            + [pltpu.SemaphoreType.DMA] * NBUF
        ),
        compiler_params=pltpu.CompilerParams(needs_layout_passes=False),
    )
    return f(text3, table2)


TR_C = 32768
TR_GRID = -(-VOCAB // TR_C)


def _tr_body(i_ref, o_ref):
    x = i_ref[...]
    eye = (lax.broadcasted_iota(jnp.int32, (64, 64), 0)
           == lax.broadcasted_iota(jnp.int32, (64, 64), 1)).astype(jnp.float32)
    y = lax.dot_general(x, eye, (((0,), (0,)), ((), ())),
                        preferred_element_type=jnp.float32)
    o_ref[:, 0:64] = y


def _transpose_table(tableT):
    return pl.pallas_call(
        _tr_body,
        grid=(TR_GRID,),
        in_specs=[pl.BlockSpec((64, TR_C), lambda i: (0, i))],
        out_specs=pl.BlockSpec((TR_C, 128), lambda i: (i, 0)),
        out_shape=jax.ShapeDtypeStruct((VOCAB, 128), jnp.float32),
    )(tableT)


def _mlp_body(x_ref, w1_ref, b1_ref, w2_ref, b2_ref, o_ref):
    x = x_ref[...]
    h = jnp.dot(x, w1_ref[...], preferred_element_type=jnp.float32) + b1_ref[...]
    h = jnp.where(h > 0, h, jnp.exp(h) - 1.0)
    o = jnp.dot(h, w2_ref[...], preferred_element_type=jnp.float32) + b2_ref[...]
    o = jnp.where(o > 0, o, jnp.exp(o) - 1.0)
    eye = (lax.broadcasted_iota(jnp.int32, (D, D), 0)
           == lax.broadcasted_iota(jnp.int32, (D, D), 1)).astype(jnp.float32)
    o_ref[...] = lax.dot_general(eye, o, (((0,), (1,)), ((), ())),
                                 precision=lax.Precision.HIGHEST,
                                 preferred_element_type=jnp.float32)


def _mlp(x, W1, b1, W2, b2):
    blk = 2048
    outT = pl.pallas_call(
        _mlp_body,
        grid=(B // blk,),
        in_specs=[
            pl.BlockSpec((blk, D), lambda i: (i, 0)),
            pl.BlockSpec((D, HID), lambda i: (0, 0)),
            pl.BlockSpec((1, HID), lambda i: (0, 0)),
            pl.BlockSpec((HID, D), lambda i: (0, 0)),
            pl.BlockSpec((1, D), lambda i: (0, 0)),
        ],
        out_specs=pl.BlockSpec((D, blk), lambda i: (0, i)),
        out_shape=jax.ShapeDtypeStruct((D, B), jnp.float32),
    )(x, W1, b1.reshape(1, HID), W2, b2.reshape(1, D))
    return outT.T


def kernel(text, emb_table, W1, b1, W2, b2):
    text3 = text.reshape(NW, NCHUNK * IDX_PER_CHUNK).astype(jnp.int32)
    table2 = _transpose_table(emb_table.T)
    pooled = _pool(text3, table2)
    return _mlp(pooled, W1, b1, W2, b2)
